# baseline (speedup 1.0000x reference)
import functools

import jax
import jax.numpy as jnp
from jax import lax
from jax.experimental import pallas as pl
from jax.experimental.pallas import tpu as pltpu
from jax.experimental.pallas import tpu_sc as plsc

V_DIM = 1_000_000
D_DIM = 64
B = 16384
L = 50
N = B * L

NC = 2
NS = 16
NW = NC * NS
PER_W = N // NW
CHUNK = 128
NCH = PER_W // CHUNK
LANES = 16
SG_COLS = 16

_mesh = plsc.VectorSubcoreMesh(core_axis_name="c", subcore_axis_name="s")


@functools.partial(
    pl.kernel,
    out_type=[
        jax.ShapeDtypeStruct((N, 2 * D_DIM), jnp.float32),
        jax.ShapeDtypeStruct((N,), jnp.float32),
    ],
    mesh=_mesh,
    scratch_types=[
        pltpu.VMEM((NCH, CHUNK), jnp.int32),
        pltpu.VMEM((CHUNK, D_DIM), jnp.float32),
        pltpu.VMEM((CHUNK,), jnp.int32),
        pltpu.VMEM((CHUNK, SG_COLS), jnp.float32),
        pltpu.VMEM((CHUNK,), jnp.float32),
        pltpu.SemaphoreType.DMA,
        pltpu.SemaphoreType.DMA,
    ],
    compiler_params=pltpu.CompilerParams(
        use_tc_tiling_on_sc=False, needs_layout_passes=False),
)
def _sc_gather(x_hbm, mu_hbm, sg_hbm, mu_out, sg_out, idx_v, mu_rows,
               idx_hi_v, sg_rows, sg_vals, mu_sem, sg_sem):
    wid = lax.axis_index("s") * NC + lax.axis_index("c")
    base = wid * PER_W
    pltpu.sync_copy(x_hbm.at[wid], idx_v)

    @pl.loop(0, NCH)
    def _chunk(c):
        idx_slice = idx_v.at[c]
        g_mu = pltpu.async_copy(mu_hbm.at[idx_slice], mu_rows, mu_sem)
        for g in range(CHUNK // LANES):
            iv = idx_v[c, pl.ds(g * LANES, LANES)]
            idx_hi_v[pl.ds(g * LANES, LANES)] = lax.shift_right_logical(iv, 4)
        g_sg = pltpu.async_copy(sg_hbm.at[idx_hi_v], sg_rows, sg_sem)
        out_base = base + c * CHUNK
        g_mu.wait()
        pltpu.sync_copy(mu_rows,
                        mu_out.at[pl.ds(out_base, CHUNK), pl.ds(0, D_DIM)])
        g_sg.wait()
        for g in range(CHUNK // LANES):
            rows = jnp.arange(LANES, dtype=jnp.int32) + (g * LANES)
            cols = idx_v[c, pl.ds(g * LANES, LANES)] & (SG_COLS - 1)
            sg_vals[pl.ds(g * LANES, LANES)] = plsc.load_gather(
                sg_rows, (rows, cols))
        pltpu.sync_copy(sg_vals, sg_out.at[pl.ds(out_base, CHUNK)])


def _softplus_body(x_ref, o_ref):
    o_ref[...] = jax.nn.softplus(x_ref[...])


def _mu_finish_body(x_ref, o_ref):
    o_ref[0, :, :] = x_ref[...][:, :D_DIM].T


_MF_BB = 512
_MF_J = B // _MF_BB


def _mu_finish(mu_pad):
    out = pl.pallas_call(
        _mu_finish_body,
        out_shape=jax.ShapeDtypeStruct((L, D_DIM, B), jnp.float32),
        grid=(L, _MF_J),
        in_specs=[pl.BlockSpec((_MF_BB, 128), lambda l, j: (l * _MF_J + j, 0))],
        out_specs=pl.BlockSpec((1, D_DIM, _MF_BB), lambda l, j: (l, 0, j)),
    )(mu_pad)
    return jnp.transpose(out, (2, 0, 1))


_SP_ROWS = N // 128
_SP_BLOCK = 800


def _softplus_tc(raw):
    x2 = raw.reshape(_SP_ROWS, 128)
    out = pl.pallas_call(
        _softplus_body,
        out_shape=jax.ShapeDtypeStruct((_SP_ROWS, 128), jnp.float32),
        grid=(_SP_ROWS // _SP_BLOCK,),
        in_specs=[pl.BlockSpec((_SP_BLOCK, 128), lambda i: (i, 0))],
        out_specs=pl.BlockSpec((_SP_BLOCK, 128), lambda i: (i, 0)),
    )(x2)
    return out


def kernel(x, mu_table, sigma_table):
    idx = x.T.astype(jnp.int32).reshape(NW, NCH, CHUNK)
    sg2 = sigma_table.reshape(V_DIM // SG_COLS, SG_COLS)
    mu_pad, sg_flat = _sc_gather(idx, mu_table, sg2)
    mu = _mu_finish(mu_pad)
    sp = _softplus_tc(sg_flat)
    sigma = jnp.transpose(sp.reshape(L, B))[:, :, None]
    return (mu, sigma)


# speedup vs baseline: 1.3339x; 1.3339x over previous
import functools

import jax
import jax.numpy as jnp
from jax import lax
from jax.experimental import pallas as pl
from jax.experimental.pallas import tpu as pltpu
from jax.experimental.pallas import tpu_sc as plsc

V_DIM = 1_000_000
D_DIM = 64
B = 16384
L = 50
N = B * L

NC = 2
NS = 16
NW = NC * NS
PER_W = N // NW
CHUNK = 128
NCH = PER_W // CHUNK
LANES = 16
SG_COLS = 16

_mesh = plsc.VectorSubcoreMesh(core_axis_name="c", subcore_axis_name="s")


@functools.partial(
    pl.kernel,
    out_type=[
        jax.ShapeDtypeStruct((N, D_DIM), jnp.float32),
        jax.ShapeDtypeStruct((N,), jnp.float32),
    ],
    mesh=_mesh,
    scratch_types=[
        pltpu.VMEM((NCH, CHUNK), jnp.int32),
        pltpu.VMEM((CHUNK, D_DIM), jnp.float32),
        pltpu.VMEM((CHUNK,), jnp.int32),
        pltpu.VMEM((CHUNK, SG_COLS), jnp.float32),
        pltpu.VMEM((CHUNK,), jnp.float32),
        pltpu.SemaphoreType.DMA,
        pltpu.SemaphoreType.DMA,
    ],
    compiler_params=pltpu.CompilerParams(
        use_tc_tiling_on_sc=False, needs_layout_passes=False),
)
def _sc_gather(x_hbm, mu_hbm, sg_hbm, mu_out, sg_out, idx_v, mu_rows,
               idx_hi_v, sg_rows, sg_vals, mu_sem, sg_sem):
    wid = lax.axis_index("s") * NC + lax.axis_index("c")
    base = wid * PER_W
    pltpu.sync_copy(x_hbm.at[wid], idx_v)

    @pl.loop(0, NCH)
    def _chunk(c):
        idx_slice = idx_v.at[c]
        g_mu = pltpu.async_copy(mu_hbm.at[idx_slice], mu_rows, mu_sem)
        for g in range(CHUNK // LANES):
            iv = idx_v[c, pl.ds(g * LANES, LANES)]
            idx_hi_v[pl.ds(g * LANES, LANES)] = lax.shift_right_logical(iv, 4)
        g_sg = pltpu.async_copy(sg_hbm.at[idx_hi_v], sg_rows, sg_sem)
        out_base = base + c * CHUNK
        g_mu.wait()
        pltpu.sync_copy(mu_rows, mu_out.at[pl.ds(out_base, CHUNK)])
        g_sg.wait()
        for g in range(CHUNK // LANES):
            rows = jnp.arange(LANES, dtype=jnp.int32) + (g * LANES)
            cols = idx_v[c, pl.ds(g * LANES, LANES)] & (SG_COLS - 1)
            sg_vals[pl.ds(g * LANES, LANES)] = plsc.load_gather(
                sg_rows, (rows, cols))
        pltpu.sync_copy(sg_vals, sg_out.at[pl.ds(out_base, CHUNK)])


def _softplus_body(x_ref, o_ref):
    o_ref[...] = jax.nn.softplus(x_ref[...])


_SP_ROWS = N // 128
_SP_BLOCK = 800


def _softplus_tc(raw):
    x2 = raw.reshape(_SP_ROWS, 128)
    out = pl.pallas_call(
        _softplus_body,
        out_shape=jax.ShapeDtypeStruct((_SP_ROWS, 128), jnp.float32),
        grid=(_SP_ROWS // _SP_BLOCK,),
        in_specs=[pl.BlockSpec((_SP_BLOCK, 128), lambda i: (i, 0))],
        out_specs=pl.BlockSpec((_SP_BLOCK, 128), lambda i: (i, 0)),
    )(x2)
    return out


def kernel(x, mu_table, sigma_table):
    idx = x.astype(jnp.int32).reshape(NW, NCH, CHUNK)
    sg2 = sigma_table.reshape(V_DIM // SG_COLS, SG_COLS)
    mu_flat, sg_flat = _sc_gather(idx, mu_table, sg2)
    mu = mu_flat.reshape(B, L, D_DIM)
    sp = _softplus_tc(sg_flat)
    sigma = sp.reshape(B, L)[:, :, None]
    return (mu, sigma)


# speedup vs baseline: 1.4506x; 1.0875x over previous
import functools

import jax
import jax.numpy as jnp
from jax import lax
from jax.experimental import pallas as pl
from jax.experimental.pallas import tpu as pltpu
from jax.experimental.pallas import tpu_sc as plsc

V_DIM = 1_000_000
D_DIM = 64
B = 16384
L = 50
N = B * L

NC = 2
NS = 16
NW = NC * NS
PER_W = N // NW
CHUNK = 128
NCH = PER_W // CHUNK
NB = 4
NG = NCH // NB
LANES = 16
SG_COLS = 16

_mesh = plsc.VectorSubcoreMesh(core_axis_name="c", subcore_axis_name="s")


@functools.partial(
    pl.kernel,
    out_type=[
        jax.ShapeDtypeStruct((N, D_DIM), jnp.float32),
        jax.ShapeDtypeStruct((N,), jnp.float32),
    ],
    mesh=_mesh,
    scratch_types=[
        pltpu.VMEM((NCH, CHUNK), jnp.int32),
        pltpu.VMEM((NB, CHUNK, D_DIM), jnp.float32),
        pltpu.VMEM((NB, CHUNK), jnp.int32),
        pltpu.VMEM((NB, CHUNK, SG_COLS), jnp.float32),
        pltpu.VMEM((CHUNK,), jnp.float32),
    ]
    + [pltpu.SemaphoreType.DMA] * (2 * NB),
    compiler_params=pltpu.CompilerParams(
        use_tc_tiling_on_sc=False, needs_layout_passes=False),
)
def _sc_gather(x_hbm, mu_hbm, sg_hbm, mu_out, sg_out, idx_v, mu_rows,
               idx_hi_v, sg_rows, sg_vals, *sems):
    mu_sems = sems[:NB]
    sg_sems = sems[NB:]
    wid = lax.axis_index("s") * NC + lax.axis_index("c")
    base = wid * PER_W
    pltpu.sync_copy(x_hbm.at[wid], idx_v)

    @pl.loop(0, NG)
    def _group(g):
        c0 = g * NB
        gathers = []
        for b in range(NB):
            c = c0 + b
            g_mu = pltpu.async_copy(mu_hbm.at[idx_v.at[c]], mu_rows.at[b],
                                    mu_sems[b])
            for q in range(CHUNK // LANES):
                iv = idx_v[c, pl.ds(q * LANES, LANES)]
                idx_hi_v[b, pl.ds(q * LANES, LANES)] = (
                    lax.shift_right_logical(iv, 4))
            g_sg = pltpu.async_copy(sg_hbm.at[idx_hi_v.at[b]], sg_rows.at[b],
                                    sg_sems[b])
            gathers.append((g_mu, g_sg))
        for b in range(NB):
            c = c0 + b
            out_base = base + c * CHUNK
            g_mu, g_sg = gathers[b]
            g_mu.wait()
            pltpu.sync_copy(mu_rows.at[b], mu_out.at[pl.ds(out_base, CHUNK)])
            g_sg.wait()
            for q in range(CHUNK // LANES):
                rows = jnp.arange(LANES, dtype=jnp.int32) + (q * LANES)
                cols = idx_v[c, pl.ds(q * LANES, LANES)] & (SG_COLS - 1)
                sg_vals[pl.ds(q * LANES, LANES)] = plsc.load_gather(
                    sg_rows.at[b], (rows, cols))
            pltpu.sync_copy(sg_vals, sg_out.at[pl.ds(out_base, CHUNK)])


def _softplus_body(x_ref, o_ref):
    o_ref[...] = jax.nn.softplus(x_ref[...])


_SP_ROWS = N // 128
_SP_BLOCK = 800


def _softplus_tc(raw):
    x2 = raw.reshape(_SP_ROWS, 128)
    out = pl.pallas_call(
        _softplus_body,
        out_shape=jax.ShapeDtypeStruct((_SP_ROWS, 128), jnp.float32),
        grid=(_SP_ROWS // _SP_BLOCK,),
        in_specs=[pl.BlockSpec((_SP_BLOCK, 128), lambda i: (i, 0))],
        out_specs=pl.BlockSpec((_SP_BLOCK, 128), lambda i: (i, 0)),
    )(x2)
    return out


def kernel(x, mu_table, sigma_table):
    idx = x.astype(jnp.int32).reshape(NW, NCH, CHUNK)
    sg2 = sigma_table.reshape(V_DIM // SG_COLS, SG_COLS)
    mu_flat, sg_flat = _sc_gather(idx, mu_table, sg2)
    mu = mu_flat.reshape(B, L, D_DIM)
    sp = _softplus_tc(sg_flat)
    sigma = sp.reshape(B, L)[:, :, None]
    return (mu, sigma)


# speedup vs baseline: 1.4712x; 1.0142x over previous
import functools

import jax
import jax.numpy as jnp
from jax import lax
from jax.experimental import pallas as pl
from jax.experimental.pallas import tpu as pltpu
from jax.experimental.pallas import tpu_sc as plsc

V_DIM = 1_000_000
D_DIM = 64
B = 16384
L = 50
N = B * L

NC = 2
NS = 16
NW = NC * NS
PER_W = N // NW
CHUNK = 128
NCH = PER_W // CHUNK
NB = 8
NG = NCH // NB
LANES = 16
SG_COLS = 16

_mesh = plsc.VectorSubcoreMesh(core_axis_name="c", subcore_axis_name="s")


@functools.partial(
    pl.kernel,
    out_type=[
        jax.ShapeDtypeStruct((N, D_DIM), jnp.float32),
        jax.ShapeDtypeStruct((N,), jnp.float32),
    ],
    mesh=_mesh,
    scratch_types=[
        pltpu.VMEM((NCH, CHUNK), jnp.int32),
        pltpu.VMEM((NB, CHUNK, D_DIM), jnp.float32),
        pltpu.VMEM((NB, CHUNK), jnp.int32),
        pltpu.VMEM((NB, CHUNK, SG_COLS), jnp.float32),
        pltpu.VMEM((CHUNK,), jnp.float32),
    ]
    + [pltpu.SemaphoreType.DMA] * (2 * NB),
    compiler_params=pltpu.CompilerParams(
        use_tc_tiling_on_sc=False, needs_layout_passes=False),
)
def _sc_gather(x_hbm, mu_hbm, sg_hbm, mu_out, sg_out, idx_v, mu_rows,
               idx_hi_v, sg_rows, sg_vals, *sems):
    mu_sems = sems[:NB]
    sg_sems = sems[NB:]
    wid = lax.axis_index("s") * NC + lax.axis_index("c")
    base = wid * PER_W
    pltpu.sync_copy(x_hbm.at[wid], idx_v)

    @pl.loop(0, NG)
    def _group(g):
        c0 = g * NB
        gathers = []
        for b in range(NB):
            c = c0 + b
            g_mu = pltpu.async_copy(mu_hbm.at[idx_v.at[c]], mu_rows.at[b],
                                    mu_sems[b])
            for q in range(CHUNK // LANES):
                iv = idx_v[c, pl.ds(q * LANES, LANES)]
                idx_hi_v[b, pl.ds(q * LANES, LANES)] = (
                    lax.shift_right_logical(iv, 4))
            g_sg = pltpu.async_copy(sg_hbm.at[idx_hi_v.at[b]], sg_rows.at[b],
                                    sg_sems[b])
            gathers.append((g_mu, g_sg))
        for b in range(NB):
            c = c0 + b
            out_base = base + c * CHUNK
            g_mu, g_sg = gathers[b]
            g_mu.wait()
            pltpu.sync_copy(mu_rows.at[b], mu_out.at[pl.ds(out_base, CHUNK)])
            g_sg.wait()
            for q in range(CHUNK // LANES):
                rows = jnp.arange(LANES, dtype=jnp.int32) + (q * LANES)
                cols = idx_v[c, pl.ds(q * LANES, LANES)] & (SG_COLS - 1)
                sg_vals[pl.ds(q * LANES, LANES)] = plsc.load_gather(
                    sg_rows.at[b], (rows, cols))
            pltpu.sync_copy(sg_vals, sg_out.at[pl.ds(out_base, CHUNK)])


def _softplus_body(x_ref, o_ref):
    o_ref[...] = jax.nn.softplus(x_ref[...])


_SP_ROWS = N // 128
_SP_BLOCK = 800


def _softplus_tc(raw):
    x2 = raw.reshape(_SP_ROWS, 128)
    out = pl.pallas_call(
        _softplus_body,
        out_shape=jax.ShapeDtypeStruct((_SP_ROWS, 128), jnp.float32),
        grid=(_SP_ROWS // _SP_BLOCK,),
        in_specs=[pl.BlockSpec((_SP_BLOCK, 128), lambda i: (i, 0))],
        out_specs=pl.BlockSpec((_SP_BLOCK, 128), lambda i: (i, 0)),
    )(x2)
    return out


def kernel(x, mu_table, sigma_table):
    idx = x.astype(jnp.int32).reshape(NW, NCH, CHUNK)
    sg2 = sigma_table.reshape(V_DIM // SG_COLS, SG_COLS)
    mu_flat, sg_flat = _sc_gather(idx, mu_table, sg2)
    mu = mu_flat.reshape(B, L, D_DIM)
    sp = _softplus_tc(sg_flat)
    sigma = sp.reshape(B, L)[:, :, None]
    return (mu, sigma)
